# Initial kernel scaffold; baseline (speedup 1.0000x reference)
import jax
import jax.numpy as jnp
from jax.experimental import pallas as pl


def kernel(x, edge_index, W1, b1, W2, b2, W3, b3):
    raise NotImplementedError("write your pallas kernel here")


# speedup vs baseline: 6.4183x; 6.4183x over previous
import functools

import jax
import jax.numpy as jnp
from jax import lax
from jax.experimental import pallas as pl
from jax.experimental.pallas import tpu as pltpu
from jax.experimental.pallas import tpu_sc as plsc

N_NODES = 10000
N_PAD = 10240
D = 128
N_EDGES = 320000
E_PAD = 327680
NW = 32
E_W = E_PAD // NW
CHUNK = 128
NCH = E_W // CHUNK

_mesh = plsc.VectorSubcoreMesh(core_axis_name="c", subcore_axis_name="s")


@functools.partial(
    pl.kernel,
    out_type=jax.ShapeDtypeStruct((2, N_PAD, 16), jnp.float32),
    mesh=_mesh,
    scratch_types=[
        pltpu.VMEM((NCH, CHUNK), jnp.int32),
        pltpu.VMEM((NCH, CHUNK), jnp.int32),
        pltpu.VMEM((CHUNK, 16), jnp.float32),
        pltpu.VMEM_SHARED((N_PAD, 16), jnp.float32),
    ],
)
def _hist(packed_hbm, out_hbm, packv, colv, buf, hist_sh):
    cid = lax.axis_index("c")
    sid = lax.axis_index("s")
    wid = sid * 2 + cid
    pltpu.sync_copy(packed_hbm.at[pl.ds(wid * NCH, NCH)], packv)

    def _cols(j, _):
        for k in range(8):
            p16 = packv[j, pl.ds(k * 16, 16)]
            colv[j, pl.ds(k * 16, 16)] = lax.bitwise_and(p16, 16383)
        return 0

    lax.fori_loop(0, NCH, _cols, 0)

    zeros16 = jnp.zeros((16,), jnp.float32)
    ones16 = jnp.full((16,), 1.0, jnp.float32)

    def _zero(i, _):
        buf[i] = zeros16
        return 0

    lax.fori_loop(0, CHUNK, _zero, 0)
    for t in range(N_PAD // 16 // CHUNK):
        pltpu.sync_copy(buf, hist_sh.at[pl.ds(sid * (N_PAD // 16) + t * CHUNK,
                                              CHUNK)])

    def _ones(i, _):
        buf[i] = ones16
        return 0

    lax.fori_loop(0, CHUNK, _ones, 0)
    plsc.subcore_barrier()

    def _count(j, _):
        pltpu.sync_copy(buf, hist_sh.at[colv.at[j]], add=True)
        return 0

    lax.fori_loop(0, NCH, _count, 0)
    plsc.subcore_barrier()

    @pl.when(sid == 0)
    def _():
        pltpu.sync_copy(hist_sh, out_hbm.at[cid])


@functools.partial(
    pl.kernel,
    out_type=jax.ShapeDtypeStruct((2, N_PAD, D), jnp.float32),
    mesh=_mesh,
    scratch_types=[
        pltpu.VMEM((NCH, CHUNK), jnp.int32),
        pltpu.VMEM((1, CHUNK), jnp.int32),
        pltpu.VMEM((NCH, CHUNK), jnp.int32),
        pltpu.VMEM((CHUNK, D), jnp.float32),
        pltpu.VMEM_SHARED((N_PAD, D), jnp.float32),
        pltpu.SemaphoreType.DMA,
    ],
)
def _agg(y_hbm, packed_hbm, out_hbm, packv, rbuf, colv, gbuf, agg_sh, sem):
    cid = lax.axis_index("c")
    sid = lax.axis_index("s")
    wid = sid * 2 + cid
    pltpu.sync_copy(packed_hbm.at[pl.ds(wid * NCH, NCH)], packv)

    def _cols(j, _):
        for k in range(8):
            p16 = packv[j, pl.ds(k * 16, 16)]
            colv[j, pl.ds(k * 16, 16)] = lax.bitwise_and(p16, 16383)
        return 0

    lax.fori_loop(0, NCH, _cols, 0)

    zeros16 = jnp.zeros((16,), jnp.float32)

    def _zero(i, _):
        for k in range(8):
            gbuf[i, pl.ds(k * 16, 16)] = zeros16
        return 0

    lax.fori_loop(0, CHUNK, _zero, 0)
    for t in range(N_PAD // 16 // CHUNK):
        pltpu.sync_copy(gbuf, agg_sh.at[pl.ds(sid * (N_PAD // 16) + t * CHUNK,
                                              CHUNK)])
    plsc.subcore_barrier()

    def _step(j, _):
        for k in range(8):
            p16 = packv[j, pl.ds(k * 16, 16)]
            rbuf[0, pl.ds(k * 16, 16)] = lax.shift_right_logical(p16, 14)
        pltpu.async_copy(y_hbm.at[rbuf.at[0]], gbuf, sem).wait()
        pltpu.sync_copy(gbuf, agg_sh.at[colv.at[j]], add=True)
        return 0

    lax.fori_loop(0, NCH, _step, 0)
    plsc.subcore_barrier()

    pltpu.sync_copy(agg_sh.at[pl.ds(sid * (N_PAD // 16), N_PAD // 16)],
                    out_hbm.at[cid].at[pl.ds(sid * (N_PAD // 16),
                                             N_PAD // 16)])


_BLK = 1280


def _dinv_body(h_ref, o_ref):
    deg = h_ref[0:80, :] + h_ref[80:160, :] + 1.0
    rid = lax.broadcasted_iota(jnp.int32, (80, 128), 0)
    qid = lax.broadcasted_iota(jnp.int32, (80, 128), 1)
    node = rid * 128 + qid
    o_ref[...] = jnp.where(node < N_NODES, lax.rsqrt(deg), 0.0)


def _lin_body(x_ref, w_ref, d_ref, o_ref):
    o_ref[...] = jnp.dot(x_ref[...], w_ref[...],
                         preferred_element_type=jnp.float32) * d_ref[...]


def _mid_body(a_ref, y_ref, d_ref, w_ref, b_ref, o_ref):
    h = d_ref[...] * (a_ref[0] + a_ref[1] + y_ref[...]) + b_ref[...]
    h = jnp.maximum(h, 0.0)
    o_ref[...] = jnp.dot(h, w_ref[...],
                         preferred_element_type=jnp.float32) * d_ref[...]


def _fin_body(a_ref, y_ref, d_ref, b_ref, o_ref):
    z = d_ref[...] * (a_ref[0] + a_ref[1] + y_ref[...]) + b_ref[...]
    m = jnp.max(z, axis=-1, keepdims=True)
    e = jnp.exp(z - m)
    s = jnp.sum(e, axis=-1, keepdims=True)
    o_ref[...] = (z - m) - jnp.log(s)


def _row_spec(blk):
    return pl.BlockSpec((blk, D), lambda i: (i, 0))


def _agg_spec(blk):
    return pl.BlockSpec((2, blk, D), lambda i: (0, i, 0))


_full_spec = pl.BlockSpec((D, D), lambda i: (0, 0))
_bias_spec = pl.BlockSpec((1, D), lambda i: (0, 0))


def kernel(x, edge_index, W1, b1, W2, b2, W3, b3):
    row = edge_index[0].astype(jnp.int32)
    col = edge_index[1].astype(jnp.int32)
    padi = jnp.full((E_PAD - N_EDGES,), N_NODES, jnp.int32)
    rowp = jnp.concatenate([row, padi])
    colp = jnp.concatenate([col, padi])
    packed = jnp.bitwise_or(jnp.left_shift(rowp, 14),
                            colp).reshape(E_PAD // CHUNK, CHUNK)
    xp = jnp.pad(x, ((0, N_PAD - N_NODES), (0, 0)))

    hist = _hist(packed)[:, :, 0]

    dinv = pl.pallas_call(
        _dinv_body,
        grid=(1,),
        in_specs=[pl.BlockSpec((160, 128), lambda i: (0, 0))],
        out_specs=pl.BlockSpec((80, 128), lambda i: (0, 0)),
        out_shape=jax.ShapeDtypeStruct((80, 128), jnp.float32),
    )(hist.reshape(160, 128))
    dinv_b = jnp.broadcast_to(dinv.reshape(N_PAD, 1), (N_PAD, D))

    grid = (N_PAD // _BLK,)
    y1 = pl.pallas_call(
        _lin_body,
        grid=grid,
        in_specs=[_row_spec(_BLK), _full_spec, _row_spec(_BLK)],
        out_specs=_row_spec(_BLK),
        out_shape=jax.ShapeDtypeStruct((N_PAD, D), jnp.float32),
    )(xp, W1, dinv_b)

    agg1 = _agg(y1, packed)

    def mid_layer(agg, y, W, b):
        return pl.pallas_call(
            _mid_body,
            grid=grid,
            in_specs=[_agg_spec(_BLK), _row_spec(_BLK), _row_spec(_BLK),
                      _full_spec, _bias_spec],
            out_specs=_row_spec(_BLK),
            out_shape=jax.ShapeDtypeStruct((N_PAD, D), jnp.float32),
        )(agg, y, dinv_b, W, b.reshape(1, D))

    y2 = mid_layer(agg1, y1, W2, b1)
    agg2 = _agg(y2, packed)
    y3 = mid_layer(agg2, y2, W3, b2)
    agg3 = _agg(y3, packed)

    out = pl.pallas_call(
        _fin_body,
        grid=grid,
        in_specs=[_agg_spec(_BLK), _row_spec(_BLK), _row_spec(_BLK),
                  _bias_spec],
        out_specs=_row_spec(_BLK),
        out_shape=jax.ShapeDtypeStruct((N_PAD, D), jnp.float32),
    )(agg3, y3, dinv_b, b3.reshape(1, D))
    return out[:N_NODES]


# speedup vs baseline: 18.1816x; 2.8328x over previous
import functools

import jax
import jax.numpy as jnp
from jax import lax
from jax.experimental import pallas as pl
from jax.experimental.pallas import tpu as pltpu
from jax.experimental.pallas import tpu_sc as plsc

N_NODES = 10000
N_PAD = 10240
D = 128
N_EDGES = 320000
E_PAD = 327680
NW = 32
E_W = E_PAD // NW
CHUNK = 128
NCH = E_W // CHUNK

_mesh = plsc.VectorSubcoreMesh(core_axis_name="c", subcore_axis_name="s")


@functools.partial(
    pl.kernel,
    out_type=jax.ShapeDtypeStruct((2, N_PAD, 16), jnp.float32),
    mesh=_mesh,
    scratch_types=[
        pltpu.VMEM((NCH, CHUNK), jnp.int32),
        pltpu.VMEM((NCH, CHUNK), jnp.int32),
        pltpu.VMEM((CHUNK, 16), jnp.float32),
        pltpu.VMEM_SHARED((N_PAD, 16), jnp.float32),
    ],
)
def _hist(packed_hbm, out_hbm, packv, colv, buf, hist_sh):
    cid = lax.axis_index("c")
    sid = lax.axis_index("s")
    wid = sid * 2 + cid
    pltpu.sync_copy(packed_hbm.at[pl.ds(wid * NCH, NCH)], packv)

    def _cols(j, _):
        for k in range(8):
            p16 = packv[j, pl.ds(k * 16, 16)]
            colv[j, pl.ds(k * 16, 16)] = lax.bitwise_and(p16, 16383)
        return 0

    lax.fori_loop(0, NCH, _cols, 0)

    zeros16 = jnp.zeros((16,), jnp.float32)
    ones16 = jnp.full((16,), 1.0, jnp.float32)

    def _zero(i, _):
        buf[i] = zeros16
        return 0

    lax.fori_loop(0, CHUNK, _zero, 0)
    for t in range(N_PAD // 16 // CHUNK):
        pltpu.sync_copy(buf, hist_sh.at[pl.ds(sid * (N_PAD // 16) + t * CHUNK,
                                              CHUNK)])

    def _ones(i, _):
        buf[i] = ones16
        return 0

    lax.fori_loop(0, CHUNK, _ones, 0)
    plsc.subcore_barrier()

    def _count(j, _):
        pltpu.sync_copy(buf, hist_sh.at[colv.at[j]], add=True)
        return 0

    lax.fori_loop(0, NCH, _count, 0)
    plsc.subcore_barrier()

    @pl.when(sid == 0)
    def _():
        pltpu.sync_copy(hist_sh, out_hbm.at[cid])


@functools.partial(
    pl.kernel,
    out_type=jax.ShapeDtypeStruct((2, N_PAD, D), jnp.float32),
    mesh=_mesh,
    scratch_types=[
        pltpu.VMEM((NCH, CHUNK), jnp.int32),
        pltpu.VMEM((2, CHUNK), jnp.int32),
        pltpu.VMEM((2, CHUNK), jnp.int32),
        pltpu.VMEM((CHUNK, D), jnp.float32),
        pltpu.VMEM((CHUNK, D), jnp.float32),
        pltpu.VMEM_SHARED((N_PAD, D), jnp.float32),
        pltpu.SemaphoreType.DMA,
        pltpu.SemaphoreType.DMA,
    ],
)
def _agg(y_hbm, packed_hbm, out_hbm, packv, ridv, cidv, g0, g1, agg_sh,
         s0, s1):
    cid = lax.axis_index("c")
    sid = lax.axis_index("s")
    wid = sid * 2 + cid
    pltpu.sync_copy(packed_hbm.at[pl.ds(wid * NCH, NCH)], packv)

    def _dec(j, slot):
        for k in range(8):
            p16 = packv[j, pl.ds(k * 16, 16)]
            ridv[slot, pl.ds(k * 16, 16)] = lax.shift_right_logical(p16, 14)
            cidv[slot, pl.ds(k * 16, 16)] = lax.bitwise_and(p16, 16383)

    zeros16 = jnp.zeros((16,), jnp.float32)

    def _zero(i, _):
        for k in range(8):
            g0[i, pl.ds(k * 16, 16)] = zeros16
        return 0

    lax.fori_loop(0, CHUNK, _zero, 0)
    for t in range(N_PAD // 16 // CHUNK):
        pltpu.sync_copy(g0, agg_sh.at[pl.ds(sid * (N_PAD // 16) + t * CHUNK,
                                            CHUNK)])
    plsc.subcore_barrier()

    def _step(j, _):
        _dec(j, 0)
        pltpu.async_copy(y_hbm.at[ridv.at[0]], g0, s0).wait()
        pltpu.sync_copy(g0, agg_sh.at[cidv.at[0]], add=True)
        return 0

    lax.fori_loop(0, NCH, _step, 0)
    plsc.subcore_barrier()

    pltpu.sync_copy(agg_sh.at[pl.ds(sid * (N_PAD // 16), N_PAD // 16)],
                    out_hbm.at[cid].at[pl.ds(sid * (N_PAD // 16),
                                             N_PAD // 16)])


_BLK = 1280


def _dinv_body(h_ref, o_ref):
    deg = h_ref[0:80, :] + h_ref[80:160, :] + 1.0
    rid = lax.broadcasted_iota(jnp.int32, (80, 128), 0)
    qid = lax.broadcasted_iota(jnp.int32, (80, 128), 1)
    node = rid * 128 + qid
    o_ref[...] = jnp.where(node < N_NODES, lax.rsqrt(deg), 0.0)


def _lin_body(x_ref, w_ref, d_ref, o_ref):
    o_ref[...] = jnp.dot(x_ref[...], w_ref[...],
                         preferred_element_type=jnp.float32) * d_ref[...]


def _mid_body(a_ref, y_ref, d_ref, w_ref, b_ref, o_ref):
    h = d_ref[...] * (a_ref[0] + a_ref[1] + y_ref[...]) + b_ref[...]
    h = jnp.maximum(h, 0.0)
    o_ref[...] = jnp.dot(h, w_ref[...],
                         preferred_element_type=jnp.float32) * d_ref[...]


def _fin_body(a_ref, y_ref, d_ref, b_ref, o_ref):
    z = d_ref[...] * (a_ref[0] + a_ref[1] + y_ref[...]) + b_ref[...]
    m = jnp.max(z, axis=-1, keepdims=True)
    e = jnp.exp(z - m)
    s = jnp.sum(e, axis=-1, keepdims=True)
    o_ref[...] = (z - m) - jnp.log(s)


def _row_spec(blk):
    return pl.BlockSpec((blk, D), lambda i: (i, 0))


def _agg_spec(blk):
    return pl.BlockSpec((2, blk, D), lambda i: (0, i, 0))


_full_spec = pl.BlockSpec((D, D), lambda i: (0, 0))
_bias_spec = pl.BlockSpec((1, D), lambda i: (0, 0))


def kernel(x, edge_index, W1, b1, W2, b2, W3, b3):
    row = edge_index[0].astype(jnp.int32)
    col = edge_index[1].astype(jnp.int32)
    padi = N_NODES + (jnp.arange(E_PAD - N_EDGES, dtype=jnp.int32)
                      % (N_PAD - N_NODES))
    rowp = jnp.concatenate([row, padi])
    colp = jnp.concatenate([col, padi])
    packed = jnp.bitwise_or(jnp.left_shift(rowp, 14),
                            colp).reshape(E_PAD // CHUNK, CHUNK)
    xp = jnp.pad(x, ((0, N_PAD - N_NODES), (0, 0)))

    hist = _hist(packed)[:, :, 0]

    dinv = pl.pallas_call(
        _dinv_body,
        grid=(1,),
        in_specs=[pl.BlockSpec((160, 128), lambda i: (0, 0))],
        out_specs=pl.BlockSpec((80, 128), lambda i: (0, 0)),
        out_shape=jax.ShapeDtypeStruct((80, 128), jnp.float32),
    )(hist.reshape(160, 128))
    dinv_b = jnp.broadcast_to(dinv.reshape(N_PAD, 1), (N_PAD, D))

    grid = (N_PAD // _BLK,)
    y1 = pl.pallas_call(
        _lin_body,
        grid=grid,
        in_specs=[_row_spec(_BLK), _full_spec, _row_spec(_BLK)],
        out_specs=_row_spec(_BLK),
        out_shape=jax.ShapeDtypeStruct((N_PAD, D), jnp.float32),
    )(xp, W1, dinv_b)

    agg1 = _agg(y1, packed)

    def mid_layer(agg, y, W, b):
        return pl.pallas_call(
            _mid_body,
            grid=grid,
            in_specs=[_agg_spec(_BLK), _row_spec(_BLK), _row_spec(_BLK),
                      _full_spec, _bias_spec],
            out_specs=_row_spec(_BLK),
            out_shape=jax.ShapeDtypeStruct((N_PAD, D), jnp.float32),
        )(agg, y, dinv_b, W, b.reshape(1, D))

    y2 = mid_layer(agg1, y1, W2, b1)
    agg2 = _agg(y2, packed)
    y3 = mid_layer(agg2, y2, W3, b2)
    agg3 = _agg(y3, packed)

    out = pl.pallas_call(
        _fin_body,
        grid=grid,
        in_specs=[_agg_spec(_BLK), _row_spec(_BLK), _row_spec(_BLK),
                  _bias_spec],
        out_specs=_row_spec(_BLK),
        out_shape=jax.ShapeDtypeStruct((N_PAD, D), jnp.float32),
    )(agg3, y3, dinv_b, b3.reshape(1, D))
    return out[:N_NODES]
